# baseline (speedup 1.0000x reference)
import functools

import jax
import jax.numpy as jnp
from jax import lax
from jax.experimental import pallas as pl
from jax.experimental.pallas import tpu as pltpu
from jax.experimental.pallas import tpu_sc as plsc


def _build_sc_kernel(M, D, B, NC, NS, L):
    NW = NC * NS
    bpw = B // NW
    n_grp = bpw // L
    W = 2 * D
    mesh = plsc.VectorSubcoreMesh(core_axis_name="c", subcore_axis_name="s")

    @functools.partial(
        pl.kernel,
        out_type=(
            jax.ShapeDtypeStruct((B // 2, W), jnp.float32),
            jax.ShapeDtypeStruct((B,), jnp.float32),
        ),
        mesh=mesh,
        compiler_params=pltpu.CompilerParams(needs_layout_passes=False),
        scratch_types=[
            pltpu.VMEM((bpw,), jnp.int32),
            pltpu.VMEM((L,), jnp.int32),
            pltpu.VMEM((bpw,), jnp.int32),
            pltpu.VMEM((bpw,), jnp.int32),
            pltpu.VMEM((bpw,), jnp.int32),
            pltpu.VMEM((bpw,), jnp.int32),
            pltpu.VMEM((bpw,), jnp.int32),
            pltpu.VMEM((bpw,), jnp.int32),
            pltpu.VMEM((2 * bpw, W), jnp.float32),
            pltpu.VMEM((bpw // 2, W), jnp.float32),
            pltpu.VMEM((bpw,), jnp.float32),
            pltpu.VMEM((bpw,), jnp.float32),
            pltpu.SemaphoreType.DMA,
            pltpu.SemaphoreType.DMA,
            pltpu.SemaphoreType.DMA,
            pltpu.SemaphoreType.DMA,
        ],
    )
    def k(mem2_h, mem_rew_h, obs2_h, rew_h, sidx_h, ivec_h,
          out2_h, out_rew_h,
          idx_v, ivec_v, pidx_v, oidx_v, rsel_v, csel_v, off_v, roff_v,
          combo_v, out_v, rew_v, obs_rew_v,
          sem0, sem1, sem2, sem3):
        wid = lax.axis_index("s") * NC + lax.axis_index("c")
        base = wid * bpw
        pltpu.sync_copy(sidx_h.at[pl.ds(base, bpw)], idx_v)
        pltpu.sync_copy(ivec_h, ivec_v)
        c_rew = pltpu.async_copy(mem_rew_h.at[idx_v], rew_v, sem1)
        iv = ivec_v[...]

        def grp(g, cnt):
            gb = pl.multiple_of(g * L, L)
            s = idx_v[pl.ds(gb, L)]
            off = s - iv
            off = jnp.where(off < 0, off + M, off)
            hit = off < B
            pidx_v[pl.ds(gb, L)] = lax.shift_right_logical(s, 1)
            oidx_v[pl.ds(gb, L)] = jnp.where(
                hit, lax.shift_right_logical(off, 1), 0)
            off_v[pl.ds(gb, L)] = off
            roff_v[pl.ds(gb, L)] = jnp.where(hit, off, 0)
            b_vec = gb + jax.lax.iota(jnp.int32, L)
            rsel_v[pl.ds(gb, L)] = jnp.where(hit, b_vec + bpw, b_vec)
            csel_v[pl.ds(gb, L)] = jnp.where(hit, off & 1, s & 1) * D
            return cnt + plsc.all_reduce_population_count(hit)

        nhit = lax.fori_loop(0, n_grp, grp, jnp.zeros((L,), jnp.int32))
        c_mem = pltpu.async_copy(
            mem2_h.at[pidx_v], combo_v.at[pl.ds(0, bpw)], sem0)
        c_mem.wait()
        c_rew.wait()

        @pl.when(nhit[0] > 0)
        def _fixup():
            pltpu.async_copy(
                obs2_h.at[oidx_v], combo_v.at[pl.ds(bpw, bpw)], sem2).wait()
            pltpu.async_copy(rew_h.at[roff_v], obs_rew_v, sem3).wait()

            def rgrp(g, c):
                gb = pl.multiple_of(g * L, L)
                off = off_v[pl.ds(gb, L)]
                hit = off < B
                rew_v[pl.ds(gb, L)] = jnp.where(
                    hit, obs_rew_v[pl.ds(gb, L)], rew_v[pl.ds(gb, L)])
                return c

            lax.fori_loop(0, n_grp, rgrp, jnp.int32(0))

        def sel(g, c):
            gb = pl.multiple_of(g * L, L)
            row = rsel_v[pl.ds(gb, L)]
            colb = csel_v[pl.ds(gb, L)]
            b_vec = gb + jax.lax.iota(jnp.int32, L)
            orow = lax.shift_right_logical(b_vec, 1)
            ocolb = (b_vec & 1) * D
            for d in range(D):
                v = plsc.load_gather(combo_v, [row, colb + d])
                plsc.store_scatter(out_v, [orow, ocolb + d], v)
            return c

        lax.fori_loop(0, n_grp, sel, jnp.int32(0))

        pltpu.sync_copy(out_v, out2_h.at[pl.ds(wid * (bpw // 2), bpw // 2)])
        pltpu.sync_copy(rew_v, out_rew_h.at[pl.ds(base, bpw)])

    return k


def kernel(memory_obs, memory_reward, obs, reward, i, sample_indices):
    M, D = memory_obs.shape
    B = obs.shape[0]
    info = plsc.get_sparse_core_info()
    NC, NS, L = info.num_cores, info.num_subcores, info.num_lanes
    mem2 = memory_obs.reshape(M // 2, 2 * D)
    obs2 = obs.reshape(B // 2, 2 * D)
    sidx = sample_indices.astype(jnp.int32)
    i_vec = jnp.full((L,), i, dtype=jnp.int32)
    k = _build_sc_kernel(M, D, B, NC, NS, L)
    out2, out_rew = k(mem2, memory_reward, obs2, reward, sidx, i_vec)
    sample_obs = out2.reshape(B, D)
    return jnp.concatenate([sample_obs, out_rew[:, None]], axis=1)


# speedup vs baseline: 7.2598x; 7.2598x over previous
import functools

import jax
import jax.numpy as jnp
from jax import lax
from jax.experimental import pallas as pl
from jax.experimental.pallas import tpu as pltpu
from jax.experimental.pallas import tpu_sc as plsc


def _build_sc_kernel(M, D, B, NC, NS, L):
    NW = NC * NS
    bpw = B // NW
    n_grp = bpw // L
    TW = 128
    KB = 8
    n_q = D // L
    mesh = plsc.VectorSubcoreMesh(core_axis_name="c", subcore_axis_name="s")

    @functools.partial(
        pl.kernel,
        out_type=(
            jax.ShapeDtypeStruct((D, B), jnp.float32),
            jax.ShapeDtypeStruct((B,), jnp.float32),
        ),
        mesh=mesh,
        compiler_params=pltpu.CompilerParams(
            use_tc_tiling_on_sc=True, needs_layout_passes=False),
        scratch_types=[
            pltpu.VMEM((bpw,), jnp.int32),
            pltpu.VMEM((L,), jnp.int32),
            pltpu.VMEM((bpw,), jnp.int32),
            pltpu.VMEM((bpw,), jnp.int32),
            pltpu.VMEM((KB, D, TW), jnp.float32),
            pltpu.VMEM((D, bpw), jnp.float32),
            pltpu.VMEM((bpw,), jnp.float32),
            pltpu.VMEM((bpw,), jnp.float32),
            pltpu.SemaphoreType.DMA,
            pltpu.SemaphoreType.DMA,
            pltpu.SemaphoreType.DMA,
        ],
    )
    def k(memT_h, mem_rew_h, obsT_h, rew_h, sidx_h, ivec_h,
          outT_h, out_rew_h,
          idx_v, ivec_v, off_v, roff_v, bufs_v, blk_v, rew_v, obs_rew_v,
          sem0, sem1, sem2):
        wid = lax.axis_index("s") * NC + lax.axis_index("c")
        base = wid * bpw
        pltpu.sync_copy(sidx_h.at[pl.ds(base, bpw)], idx_v)
        pltpu.sync_copy(ivec_h, ivec_v)
        c_rew = pltpu.async_copy(mem_rew_h.at[idx_v], rew_v, sem1)
        iv = ivec_v[...]

        def grp(g, cnt):
            gb = pl.multiple_of(g * L, L)
            s = idx_v[pl.ds(gb, L)]
            off = s - iv
            off = jnp.where(off < 0, off + M, off)
            hit = off < B
            off_v[pl.ds(gb, L)] = off
            roff_v[pl.ds(gb, L)] = jnp.where(hit, off, 0)
            return cnt + plsc.all_reduce_population_count(hit)

        nhit = lax.fori_loop(0, n_grp, grp, jnp.zeros((L,), jnp.int32))

        def _extract_col(src_ref, col, dst_col):
            cvec = jnp.full((L,), col, dtype=jnp.int32)
            dvec = jnp.full((L,), dst_col, dtype=jnp.int32)
            for q in range(n_q):
                r_vec = q * L + lax.iota(jnp.int32, L)
                v = plsc.load_gather(src_ref, [r_vec, cvec])
                plsc.store_scatter(blk_v, [r_vec, dvec], v)

        def grp_gather(g, c):
            gb = pl.multiple_of(g * L, L)
            s = idx_v[pl.ds(gb, L)]
            for h in range(L // KB):
                copies = []
                for j in range(KB):
                    sj = s[h * KB + j]
                    c0 = pl.multiple_of(
                        lax.shift_left(lax.shift_right_logical(sj, 7), 7), TW)
                    copies.append(pltpu.async_copy(
                        memT_h.at[:, pl.ds(c0, TW)], bufs_v.at[j], sem0))
                for cp in copies:
                    cp.wait()
                for j in range(KB):
                    sj = s[h * KB + j]
                    _extract_col(bufs_v.at[j], sj & (TW - 1), gb + h * KB + j)
            return c

        lax.fori_loop(0, n_grp, grp_gather, jnp.int32(0))
        c_rew.wait()

        @pl.when(nhit[0] > 0)
        def _fixup():
            pltpu.async_copy(rew_h.at[roff_v], obs_rew_v, sem2).wait()

            def rfix(g, c):
                gb = pl.multiple_of(g * L, L)
                off = off_v[pl.ds(gb, L)]
                hit = off < B
                rew_v[pl.ds(gb, L)] = jnp.where(
                    hit, obs_rew_v[pl.ds(gb, L)], rew_v[pl.ds(gb, L)])
                for j in range(L):
                    oj = off[j]
                    @pl.when(oj < B)
                    def _(j=j, oj=oj):
                        c0 = pl.multiple_of(
                            lax.shift_left(lax.shift_right_logical(oj, 7), 7),
                            TW)
                        pltpu.sync_copy(
                            obsT_h.at[:, pl.ds(c0, TW)], bufs_v.at[0])
                        _extract_col(bufs_v.at[0], oj & (TW - 1), gb + j)
                return c

            lax.fori_loop(0, n_grp, rfix, jnp.int32(0))

        pltpu.sync_copy(blk_v, outT_h.at[:, pl.ds(base, bpw)])
        pltpu.sync_copy(rew_v, out_rew_h.at[pl.ds(base, bpw)])

    return k


def kernel(memory_obs, memory_reward, obs, reward, i, sample_indices):
    M, D = memory_obs.shape
    B = obs.shape[0]
    info = plsc.get_sparse_core_info()
    NC, NS, L = info.num_cores, info.num_subcores, info.num_lanes
    memT = memory_obs.T
    obsT = obs.T
    sidx = sample_indices.astype(jnp.int32)
    i_vec = jnp.full((L,), i, dtype=jnp.int32)
    k = _build_sc_kernel(M, D, B, NC, NS, L)
    outT, out_rew = k(memT, memory_reward, obsT, reward, sidx, i_vec)
    return jnp.concatenate([outT.T, out_rew[:, None]], axis=1)


# speedup vs baseline: 8.0414x; 1.1077x over previous
import functools

import jax
import jax.numpy as jnp
from jax import lax
from jax.experimental import pallas as pl
from jax.experimental.pallas import tpu as pltpu
from jax.experimental.pallas import tpu_sc as plsc


def _build_sc_kernel(M, D, B, NC, NS, L):
    NW = NC * NS
    bpw = B // NW
    n_grp = bpw // L
    TW = 128
    KB = 4
    n_b = bpw // KB
    n_q = D // L
    mesh = plsc.VectorSubcoreMesh(core_axis_name="c", subcore_axis_name="s")

    @functools.partial(
        pl.kernel,
        out_type=jax.ShapeDtypeStruct((D + 1, B), jnp.float32),
        mesh=mesh,
        compiler_params=pltpu.CompilerParams(
            use_tc_tiling_on_sc=True, needs_layout_passes=False),
        scratch_types=[
            pltpu.VMEM((bpw,), jnp.int32),
            pltpu.VMEM((L,), jnp.int32),
            pltpu.VMEM((bpw,), jnp.int32),
            pltpu.VMEM((bpw,), jnp.int32),
            pltpu.VMEM((2, KB, D, TW), jnp.float32),
            pltpu.VMEM((D, bpw), jnp.float32),
            pltpu.VMEM((1, bpw), jnp.float32),
            pltpu.VMEM((bpw,), jnp.float32),
            pltpu.VMEM((bpw,), jnp.float32),
            pltpu.SemaphoreType.DMA,
            pltpu.SemaphoreType.DMA,
            pltpu.SemaphoreType.DMA,
            pltpu.SemaphoreType.DMA,
        ],
    )
    def k(memT_h, mem_rew_h, obsT_h, rew_h, sidx_h, ivec_h,
          outT_h,
          idx_v, ivec_v, off_v, roff_v, bufs_v, blk_v, rrow_v, rew_v,
          obs_rew_v, semA, semB, sem1, sem2):
        wid = lax.axis_index("s") * NC + lax.axis_index("c")
        base = wid * bpw
        pltpu.sync_copy(sidx_h.at[pl.ds(base, bpw)], idx_v)
        pltpu.sync_copy(ivec_h, ivec_v)
        c_rew = pltpu.async_copy(mem_rew_h.at[idx_v], rew_v, sem1)
        iv = ivec_v[...]

        def grp(g, cnt):
            gb = pl.multiple_of(g * L, L)
            s = idx_v[pl.ds(gb, L)]
            off = s - iv
            off = jnp.where(off < 0, off + M, off)
            hit = off < B
            off_v[pl.ds(gb, L)] = off
            roff_v[pl.ds(gb, L)] = jnp.where(hit, off, 0)
            return cnt + plsc.all_reduce_population_count(hit)

        nhit = lax.fori_loop(0, n_grp, grp, jnp.zeros((L,), jnp.int32))

        lane16 = lax.iota(jnp.int32, L)

        def _batch_idx(b):
            pos = jnp.minimum(b * KB + lane16, bpw - 1)
            return plsc.load_gather(idx_v, [pos])

        def _fire(b, ring, sem):
            sv = _batch_idx(b)
            for j in range(KB):
                c0 = pl.multiple_of(
                    lax.shift_left(lax.shift_right_logical(sv[j], 7), 7), TW)
                pltpu.async_copy(
                    memT_h.at[:, pl.ds(c0, TW)], bufs_v.at[ring, j], sem)

        def _drain(ring, sem):
            for j in range(KB):
                pltpu.make_async_copy(
                    memT_h.at[:, pl.ds(0, TW)], bufs_v.at[ring, j], sem).wait()

        def _extract_col(src_ref, col, dst_col):
            cvec = jnp.full((L,), col, dtype=jnp.int32)
            dvec = jnp.full((L,), dst_col, dtype=jnp.int32)
            for q in range(n_q):
                r_vec = q * L + lane16
                v = plsc.load_gather(src_ref, [r_vec, cvec])
                plsc.store_scatter(blk_v, [r_vec, dvec], v)

        def _extract(b, ring):
            sv = _batch_idx(b)
            for j in range(KB):
                _extract_col(bufs_v.at[ring, j], sv[j] & (TW - 1), b * KB + j)

        _fire(jnp.int32(0), 0, semA)

        def pipe(t, c):
            b0 = t * 2
            _fire(b0 + 1, 1, semB)
            _drain(0, semA)
            _extract(b0, 0)

            @pl.when(t < n_b // 2 - 1)
            def _():
                _fire(b0 + 2, 0, semA)

            _drain(1, semB)
            _extract(b0 + 1, 1)
            return c

        lax.fori_loop(0, n_b // 2, pipe, jnp.int32(0))
        c_rew.wait()

        @pl.when(nhit[0] > 0)
        def _fixup():
            pltpu.async_copy(rew_h.at[roff_v], obs_rew_v, sem2).wait()

            def rfix(g, c):
                gb = pl.multiple_of(g * L, L)
                off = off_v[pl.ds(gb, L)]
                hit = off < B
                rew_v[pl.ds(gb, L)] = jnp.where(
                    hit, obs_rew_v[pl.ds(gb, L)], rew_v[pl.ds(gb, L)])
                for j in range(L):
                    oj = off[j]
                    @pl.when(oj < B)
                    def _(j=j, oj=oj):
                        c0 = pl.multiple_of(
                            lax.shift_left(lax.shift_right_logical(oj, 7), 7),
                            TW)
                        pltpu.sync_copy(
                            obsT_h.at[:, pl.ds(c0, TW)], bufs_v.at[0, 0])
                        _extract_col(bufs_v.at[0, 0], oj & (TW - 1), gb + j)
                return c

            lax.fori_loop(0, n_grp, rfix, jnp.int32(0))

        def rrow(g, c):
            gb = pl.multiple_of(g * L, L)
            rrow_v[0, pl.ds(gb, L)] = rew_v[pl.ds(gb, L)]
            return c

        lax.fori_loop(0, n_grp, rrow, jnp.int32(0))

        pltpu.sync_copy(blk_v, outT_h.at[pl.ds(0, D), pl.ds(base, bpw)])
        pltpu.sync_copy(rrow_v, outT_h.at[pl.ds(D, 1), pl.ds(base, bpw)])

    return k


def kernel(memory_obs, memory_reward, obs, reward, i, sample_indices):
    M, D = memory_obs.shape
    B = obs.shape[0]
    info = plsc.get_sparse_core_info()
    NC, NS, L = info.num_cores, info.num_subcores, info.num_lanes
    memT = memory_obs.T
    obsT = obs.T
    sidx = sample_indices.astype(jnp.int32)
    i_vec = jnp.full((L,), i, dtype=jnp.int32)
    k = _build_sc_kernel(M, D, B, NC, NS, L)
    outT = k(memT, memory_reward, obsT, reward, sidx, i_vec)
    return outT.T
